# Initial kernel scaffold; baseline (speedup 1.0000x reference)
import jax
import jax.numpy as jnp
from jax.experimental import pallas as pl


def kernel(features, label, emb_mean, emb_std, fc_small, fc_concat, log_alpha, rand_array):
    raise NotImplementedError("write your pallas kernel here")


# speedup vs baseline: 32.1364x; 32.1364x over previous
import jax
import jax.numpy as jnp
import numpy as np
from jax.experimental import pallas as pl
from jax.experimental.pallas import tpu as pltpu

N_COLS = 22
N_PAIRS = N_COLS * (N_COLS - 1) // 2
EMB_NUM = 12
D = 64
N_OPS = 5
TILE = 1024

_I_IDX, _J_IDX = np.triu_indices(N_COLS, k=1)
_MI = np.zeros((N_COLS, N_PAIRS), np.float32)
_MJ = np.zeros((N_COLS, N_PAIRS), np.float32)
_MI[_I_IDX, np.arange(N_PAIRS)] = 1.0
_MJ[_J_IDX, np.arange(N_PAIRS)] = 1.0


def _nasp_kernel(la_ref, feat_ref, mean_ref, std_ref, v_ref,
                 f0_ref, f1_ref, f2_ref, f3_ref, fa_ref, fb_ref,
                 mi_ref, mj_ref, out_ref, acc0_ref, acc1_ref):
    logits = [la_ref[k] for k in range(N_OPS)]
    best = logits[0]
    sel = jnp.int32(0)
    for k in range(1, N_OPS):
        is_new = logits[k] > best
        best = jnp.where(is_new, logits[k], best)
        sel = jnp.where(is_new, jnp.int32(k), sel)
    w = [(sel == k).astype(jnp.float32) for k in range(N_OPS)]
    w0, w1, w2, w3, w4 = w
    hw2 = 0.5 * w2
    hw3 = 0.5 * w3

    v = v_ref[...]
    iota12 = jax.lax.broadcasted_iota(jnp.int32, (1, EMB_NUM), 1)
    cols = []
    for c in range(N_COLS):
        oh = (feat_ref[:, c:c + 1] == iota12).astype(jnp.float32)
        mu = jnp.dot(oh, mean_ref[c], preferred_element_type=jnp.float32,
                     precision=jax.lax.Precision.HIGHEST)
        sd = jnp.dot(oh, std_ref[c], preferred_element_type=jnp.float32,
                     precision=jax.lax.Precision.HIGHEST)
        cols.append(mu + jnp.log1p(jnp.exp(sd)) * v * 0.01)

    s_w = w0 * f0_ref[...] + hw2 * f2_ref[...] + hw3 * f3_ref[...]
    gi = s_w + w4 * fa_ref[...]
    gj = s_w + w4 * fb_ref[...]
    wcol = (jnp.dot(mi_ref[...], gi, preferred_element_type=jnp.float32,
                    precision=jax.lax.Precision.HIGHEST) +
            jnp.dot(mj_ref[...], gj, preferred_element_type=jnp.float32,
                    precision=jax.lax.Precision.HIGHEST))
    a0 = cols[0] * wcol[0:1, :D]
    a1 = cols[0] * wcol[0:1, D:]
    for c in range(1, N_COLS):
        a0 = a0 + cols[c] * wcol[c:c + 1, :D]
        a1 = a1 + cols[c] * wcol[c:c + 1, D:]
    acc0_ref[...] = a0
    acc1_ref[...] = a1

    @pl.when(w1 + w2 + w3 > 0.5)
    def _pairwise():
        p0 = acc0_ref[...]
        p1 = acc1_ref[...]
        for p in range(N_PAIRS):
            i = int(_I_IDX[p])
            j = int(_J_IDX[p])
            ei = cols[i]
            ej = cols[j]
            m = ei * ej
            a = jnp.abs(ei - ej)
            wm = w1 * f1_ref[p:p + 1, :]
            wa = hw2 * f2_ref[p:p + 1, :] - hw3 * f3_ref[p:p + 1, :]
            p0 = p0 + m * wm[:, :D] + a * wa[:, :D]
            p1 = p1 + m * wm[:, D:] + a * wa[:, D:]
        acc0_ref[...] = p0
        acc1_ref[...] = p1

    r0 = jnp.sum(acc0_ref[...], axis=1, keepdims=True)
    r1 = jnp.sum(acc1_ref[...], axis=1, keepdims=True)
    out_ref[...] = jnp.concatenate([r0, r1], axis=1)


def kernel(features, label, emb_mean, emb_std, fc_small, fc_concat, log_alpha, rand_array):
    del label
    B = features.shape[1]
    featT = features.T.astype(jnp.int32)
    v = rand_array[:B * D].reshape(B, D)
    f0 = fc_small[:, 0].reshape(N_PAIRS, 2 * D)
    f1 = fc_small[:, 1].reshape(N_PAIRS, 2 * D)
    f2 = fc_small[:, 2].reshape(N_PAIRS, 2 * D)
    f3 = fc_small[:, 3].reshape(N_PAIRS, 2 * D)
    fa = fc_concat[:, :, :D].reshape(N_PAIRS, 2 * D)
    fb = fc_concat[:, :, D:].reshape(N_PAIRS, 2 * D)
    la = log_alpha.reshape(N_OPS)
    mi = jnp.asarray(_MI)
    mj = jnp.asarray(_MJ)

    full2 = lambda t: (0, 0)
    out = pl.pallas_call(
        _nasp_kernel,
        grid=(B // TILE,),
        in_specs=[
            pl.BlockSpec(memory_space=pltpu.SMEM),
            pl.BlockSpec((TILE, N_COLS), lambda t: (t, 0)),
            pl.BlockSpec((N_COLS, EMB_NUM, D), lambda t: (0, 0, 0)),
            pl.BlockSpec((N_COLS, EMB_NUM, D), lambda t: (0, 0, 0)),
            pl.BlockSpec((TILE, D), lambda t: (t, 0)),
            pl.BlockSpec((N_PAIRS, 2 * D), full2),
            pl.BlockSpec((N_PAIRS, 2 * D), full2),
            pl.BlockSpec((N_PAIRS, 2 * D), full2),
            pl.BlockSpec((N_PAIRS, 2 * D), full2),
            pl.BlockSpec((N_PAIRS, 2 * D), full2),
            pl.BlockSpec((N_PAIRS, 2 * D), full2),
            pl.BlockSpec((N_COLS, N_PAIRS), full2),
            pl.BlockSpec((N_COLS, N_PAIRS), full2),
        ],
        out_specs=pl.BlockSpec((TILE, 2), lambda t: (t, 0)),
        out_shape=jax.ShapeDtypeStruct((B, 2), jnp.float32),
        scratch_shapes=[pltpu.VMEM((TILE, D), jnp.float32),
                        pltpu.VMEM((TILE, D), jnp.float32)],
    )(la, featT, emb_mean, emb_std, v, f0, f1, f2, f3, fa, fb, mi, mj)
    return out


# speedup vs baseline: 74.5403x; 2.3195x over previous
import jax
import jax.numpy as jnp
import numpy as np
from jax.experimental import pallas as pl
from jax.experimental.pallas import tpu as pltpu

N_COLS = 22
N_PAIRS = N_COLS * (N_COLS - 1) // 2
EMB_NUM = 12
ROWS = N_COLS * EMB_NUM
D = 64
N_OPS = 5
TILE = 1024

_I_IDX, _J_IDX = np.triu_indices(N_COLS, k=1)
_MIR = np.zeros((ROWS, N_PAIRS), np.float32)
_MJR = np.zeros((ROWS, N_PAIRS), np.float32)
for _p in range(N_PAIRS):
    _MIR[_I_IDX[_p] * EMB_NUM:(_I_IDX[_p] + 1) * EMB_NUM, _p] = 1.0
    _MJR[_J_IDX[_p] * EMB_NUM:(_J_IDX[_p] + 1) * EMB_NUM, _p] = 1.0

_HI = jax.lax.Precision.HIGHEST


def _nasp_kernel(la_ref, pi_ref, pj_ref, featr_ref, mean_ref, std_ref, v_ref,
                 f0_ref, f1_ref, f2_ref, f3_ref, fa_ref, fb_ref,
                 f1p_ref, f2p_ref, f3p_ref,
                 mir_ref, mjr_ref, out_ref, e_ref, acc0_ref, acc1_ref):
    logits = [la_ref[k] for k in range(N_OPS)]
    best = logits[0]
    sel = jnp.int32(0)
    for k in range(1, N_OPS):
        is_new = logits[k] > best
        best = jnp.where(is_new, logits[k], best)
        sel = jnp.where(is_new, jnp.int32(k), sel)
    w = [(sel == k).astype(jnp.float32) for k in range(N_OPS)]
    w0, w1, w2, w3, w4 = w
    hw2 = 0.5 * w2
    hw3 = 0.5 * w3

    sp01 = jnp.log1p(jnp.exp(std_ref[...])) * 0.01

    s_w = w0 * f0_ref[...] + hw2 * f2_ref[...] + hw3 * f3_ref[...]
    gi = s_w + w4 * fa_ref[...]
    gj = s_w + w4 * fb_ref[...]
    wcolr = (jnp.dot(mir_ref[...], gi, preferred_element_type=jnp.float32,
                     precision=_HI) +
             jnp.dot(mjr_ref[...], gj, preferred_element_type=jnp.float32,
                     precision=_HI))

    iota = jax.lax.broadcasted_iota(jnp.int32, (1, ROWS), 1)
    pattern = jax.lax.rem(iota, jnp.int32(EMB_NUM))
    oh = (featr_ref[...] == pattern).astype(jnp.float32)

    mean = mean_ref[...]
    p0 = jnp.sum(mean * wcolr[:, :D], axis=1, keepdims=True)
    p1 = jnp.sum(mean * wcolr[:, D:], axis=1, keepdims=True)
    pbig = jnp.concatenate([p0, p1], axis=1)
    sbig = jnp.concatenate([sp01, sp01], axis=1) * wcolr
    lin = jnp.dot(oh, pbig, preferred_element_type=jnp.float32, precision=_HI)
    g = jnp.dot(oh, sbig, preferred_element_type=jnp.float32, precision=_HI)
    v = v_ref[...]
    h = g * jnp.concatenate([v, v], axis=1)
    lin0 = lin[:, 0:1] + jnp.sum(h[:, :D], axis=1, keepdims=True)
    lin1 = lin[:, 1:2] + jnp.sum(h[:, D:], axis=1, keepdims=True)

    acc0_ref[...] = jnp.zeros((TILE, D), jnp.float32)
    acc1_ref[...] = jnp.zeros((TILE, D), jnp.float32)

    @pl.when(w1 + w2 + w3 > 0.5)
    def _pairwise():
        for c in range(N_COLS):
            ohc = oh[:, c * EMB_NUM:(c + 1) * EMB_NUM]
            mu = jnp.dot(ohc, mean_ref[c * EMB_NUM:(c + 1) * EMB_NUM, :],
                         preferred_element_type=jnp.float32, precision=_HI)
            sp = jnp.dot(ohc, sp01[c * EMB_NUM:(c + 1) * EMB_NUM, :],
                         preferred_element_type=jnp.float32, precision=_HI)
            e_ref[c] = mu + sp * v
        def body(p, carry):
            i = pi_ref[p]
            j = pj_ref[p]
            ei = e_ref[i]
            ej = e_ref[j]
            m = ei * ej
            a = jnp.abs(ei - ej)
            wm = w1 * f1p_ref[p]
            wa = hw2 * f2p_ref[p] - hw3 * f3p_ref[p]
            acc0_ref[...] += m * wm[:, :D] + a * wa[:, :D]
            acc1_ref[...] += m * wm[:, D:] + a * wa[:, D:]
            return carry

        jax.lax.fori_loop(0, N_PAIRS, body, 0)

    r0 = lin0 + jnp.sum(acc0_ref[...], axis=1, keepdims=True)
    r1 = lin1 + jnp.sum(acc1_ref[...], axis=1, keepdims=True)
    out_ref[...] = jnp.concatenate([r0, r1], axis=1)


def kernel(features, label, emb_mean, emb_std, fc_small, fc_concat, log_alpha, rand_array):
    del label
    B = features.shape[1]
    featr = jnp.repeat(features.T.astype(jnp.int32), EMB_NUM, axis=1)
    v = rand_array[:B * D].reshape(B, D)
    mean264 = emb_mean.reshape(ROWS, D)
    std264 = emb_std.reshape(ROWS, D)
    f0 = fc_small[:, 0].reshape(N_PAIRS, 2 * D)
    f1 = fc_small[:, 1].reshape(N_PAIRS, 2 * D)
    f2 = fc_small[:, 2].reshape(N_PAIRS, 2 * D)
    f3 = fc_small[:, 3].reshape(N_PAIRS, 2 * D)
    fa = fc_concat[:, :, :D].reshape(N_PAIRS, 2 * D)
    fb = fc_concat[:, :, D:].reshape(N_PAIRS, 2 * D)
    la = log_alpha.reshape(N_OPS)
    mir = jnp.asarray(_MIR)
    mjr = jnp.asarray(_MJR)
    pidx = jnp.asarray(_I_IDX.astype(np.int32))
    pjdx = jnp.asarray(_J_IDX.astype(np.int32))
    f1p = f1.reshape(N_PAIRS, 1, 2 * D)
    f2p = f2.reshape(N_PAIRS, 1, 2 * D)
    f3p = f3.reshape(N_PAIRS, 1, 2 * D)

    full2 = lambda t: (0, 0)
    out = pl.pallas_call(
        _nasp_kernel,
        grid=(B // TILE,),
        in_specs=[
            pl.BlockSpec(memory_space=pltpu.SMEM),
            pl.BlockSpec(memory_space=pltpu.SMEM),
            pl.BlockSpec(memory_space=pltpu.SMEM),
            pl.BlockSpec((TILE, ROWS), lambda t: (t, 0)),
            pl.BlockSpec((ROWS, D), full2),
            pl.BlockSpec((ROWS, D), full2),
            pl.BlockSpec((TILE, D), lambda t: (t, 0)),
            pl.BlockSpec((N_PAIRS, 2 * D), full2),
            pl.BlockSpec((N_PAIRS, 2 * D), full2),
            pl.BlockSpec((N_PAIRS, 2 * D), full2),
            pl.BlockSpec((N_PAIRS, 2 * D), full2),
            pl.BlockSpec((N_PAIRS, 2 * D), full2),
            pl.BlockSpec((N_PAIRS, 2 * D), full2),
            pl.BlockSpec((N_PAIRS, 1, 2 * D), lambda t: (0, 0, 0)),
            pl.BlockSpec((N_PAIRS, 1, 2 * D), lambda t: (0, 0, 0)),
            pl.BlockSpec((N_PAIRS, 1, 2 * D), lambda t: (0, 0, 0)),
            pl.BlockSpec((ROWS, N_PAIRS), full2),
            pl.BlockSpec((ROWS, N_PAIRS), full2),
        ],
        out_specs=pl.BlockSpec((TILE, 2), lambda t: (t, 0)),
        out_shape=jax.ShapeDtypeStruct((B, 2), jnp.float32),
        scratch_shapes=[pltpu.VMEM((N_COLS, TILE, D), jnp.float32),
                        pltpu.VMEM((TILE, D), jnp.float32),
                        pltpu.VMEM((TILE, D), jnp.float32)],
    )(la, pidx, pjdx, featr, mean264, std264, v, f0, f1, f2, f3, fa, fb,
      f1p, f2p, f3p, mir, mjr)
    return out
